# baseline (speedup 1.0000x reference)
import functools

import jax
import jax.numpy as jnp
from jax import lax
from jax.experimental import pallas as pl
from jax.experimental.pallas import tpu as pltpu
from jax.experimental.pallas import tpu_sc as plsc

_N = 100000
_D = 500
_MD = 100
_B = 16384

_NC = 2
_NS = 16
_NW = _NC * _NS
_BPW = _B // _NW
_CH = 64
_NCH = _BPW // _CH


_PIECES = ((0, 0), (128, 128), (256, 256), (384, 384))


def _sc_gather_body(mem_hbm, tail_hbm, rm_hbm, idx_hbm, w_hbm, h_out, rm2_out,
                    idx_v, w_v, hbuf, rmbuf, sem, sem2):
    wid = lax.axis_index("s") * _NC + lax.axis_index("c")
    base = wid * _BPW
    pltpu.sync_copy(idx_hbm.at[pl.ds(base, _BPW)], idx_v)
    pltpu.sync_copy(w_hbm.at[pl.ds(base, _BPW)], w_v)
    for c in range(_NCH):
        ids = idx_v.at[pl.ds(c * _CH, _CH)]
        cps = [pltpu.async_copy(
                   mem_hbm.at[ids, pl.ds(src, 128)], hbuf.at[k], sem)
               for k, (src, _) in enumerate(_PIECES[:3])]
        cps.append(pltpu.async_copy(tail_hbm.at[ids], hbuf.at[3], sem))
        cp_m = pltpu.async_copy(
            rm_hbm.at[w_v.at[pl.ds(c * _CH, _CH)]], rmbuf, sem2)
        rows = pl.ds(base + c * _CH, _CH)
        for k, (_, dst) in enumerate(_PIECES):
            cps[k].wait()
            pltpu.sync_copy(hbuf.at[k], h_out.at[rows, pl.ds(dst, 128)])
        cp_m.wait()
        pltpu.sync_copy(rmbuf, rm2_out.at[rows])


@functools.cache
def _sc_gather():
    return pl.kernel(
        _sc_gather_body,
        out_type=[jax.ShapeDtypeStruct((_B, 512), jnp.float32),
                  jax.ShapeDtypeStruct((_B, 128), jnp.float32)],
        mesh=plsc.VectorSubcoreMesh(core_axis_name="c", subcore_axis_name="s",
                                    num_cores=_NC, num_subcores=_NS),
        scratch_types=[
            pltpu.VMEM((_BPW,), jnp.int32),
            pltpu.VMEM((_BPW,), jnp.int32),
            pltpu.VMEM((4, _CH, 128), jnp.float32),
            pltpu.VMEM((_CH, 128), jnp.float32),
            pltpu.SemaphoreType.DMA,
            pltpu.SemaphoreType.DMA,
        ],
    )


_BM = 256


def _tc_body(h_ref, rm_ref, w1_ref, b1_ref, w2_ref, b2_ref,
             wir_ref, wiz_ref, win_ref, bi_ref,
             whr_ref, whz_ref, whn_ref, bh_ref, out_ref):
    f32 = jnp.float32
    bf16 = jnp.bfloat16
    cdims = (((1,), (1,)), ((), ()))
    h = h_ref[:, :_D]
    hb = h.astype(bf16)
    m = jax.nn.relu(
        lax.dot_general(rm_ref[:, :_MD], w1_ref[...], cdims,
                        preferred_element_type=f32) + b1_ref[...])
    m = lax.dot_general(m, w2_ref[...], cdims,
                        preferred_element_type=f32) + b2_ref[...]
    mb = m.astype(bf16)
    gir = lax.dot_general(mb, wir_ref[...], cdims,
                          preferred_element_type=f32) + bi_ref[0:1, :]
    giz = lax.dot_general(mb, wiz_ref[...], cdims,
                          preferred_element_type=f32) + bi_ref[1:2, :]
    gin = lax.dot_general(mb, win_ref[...], cdims,
                          preferred_element_type=f32) + bi_ref[2:3, :]
    ghr = lax.dot_general(hb, whr_ref[...], cdims,
                          preferred_element_type=f32) + bh_ref[0:1, :]
    ghz = lax.dot_general(hb, whz_ref[...], cdims,
                          preferred_element_type=f32) + bh_ref[1:2, :]
    ghn = lax.dot_general(hb, whn_ref[...], cdims,
                          preferred_element_type=f32) + bh_ref[2:3, :]
    r = jax.nn.sigmoid(gir + ghr)
    z = jax.nn.sigmoid(giz + ghz)
    n = jnp.tanh(gin + r * ghn)
    out_ref[...] = (1.0 - z) * n + z * h


def _tc_call(h, rm2, W1, b1, W2, b2, W_ih, b_ih, W_hh, b_hh):
    bf16 = jnp.bfloat16
    W_ih = W_ih.astype(bf16)
    W_hh = W_hh.astype(bf16)
    wir, wiz, win = W_ih[:_D], W_ih[_D:2 * _D], W_ih[2 * _D:]
    whr, whz, whn = W_hh[:_D], W_hh[_D:2 * _D], W_hh[2 * _D:]
    bi = b_ih.reshape(3, _D)
    bh = b_hh.reshape(3, _D)
    full = lambda s: pl.BlockSpec(s, lambda i: (0, 0))
    return pl.pallas_call(
        _tc_body,
        grid=(_B // _BM,),
        in_specs=[
            pl.BlockSpec((_BM, 512), lambda i: (i, 0)),
            pl.BlockSpec((_BM, 128), lambda i: (i, 0)),
            full((_MD // 2, _MD)), full((1, _MD // 2)),
            full((_MD, _MD // 2)), full((1, _MD)),
            full((_D, _MD)), full((_D, _MD)), full((_D, _MD)), full((3, _D)),
            full((_D, _D)), full((_D, _D)), full((_D, _D)), full((3, _D)),
        ],
        out_specs=pl.BlockSpec((_BM, _D), lambda i: (i, 0)),
        out_shape=jax.ShapeDtypeStruct((_B, _D), jnp.float32),
    )(h, rm2, W1, b1.reshape(1, -1), W2, b2.reshape(1, -1),
      wir, wiz, win, bi, whr, whz, whn, bh)


def kernel(memory, node_idx, raw_messages, W1, b1, W2, b2,
           W_ih, b_ih, W_hh, b_hh):
    idx = node_idx.astype(jnp.int32)
    w = jnp.zeros((_N,), jnp.int32).at[idx].set(
        jnp.arange(_B, dtype=jnp.int32))[idx]
    rm_p = jnp.pad(raw_messages, ((0, 0), (0, 128 - _MD)))
    tail = jnp.pad(lax.slice(memory, (0, 384), (_N, _D)),
                   ((0, 0), (0, 12)))
    h, rm2 = _sc_gather()(memory, tail, rm_p, idx, w)
    return _tc_call(h, rm2, W1, b1, W2, b2, W_ih, b_ih, W_hh, b_hh)


# speedup vs baseline: 1.0445x; 1.0445x over previous
import functools

import jax
import jax.numpy as jnp
from jax import lax
from jax.experimental import pallas as pl
from jax.experimental.pallas import tpu as pltpu
from jax.experimental.pallas import tpu_sc as plsc

_N = 100000
_D = 500
_MD = 100
_B = 16384

_NC = 2
_NS = 16
_NW = _NC * _NS
_BPW = _B // _NW
_CH = 32
_NCH = _BPW // _CH
_SCAN = 1024


_PIECES = ((0, 0), (128, 128), (256, 256), (384, 384))


def _sc_gather_body(mem_hbm, tail_hbm, rm_hbm, idx_hbm, h_out, rm2_out,
                    idx_v, w_v, tab, scan_v, hbuf, rmbuf, sem, sem2):
    wid = lax.axis_index("s") * _NC + lax.axis_index("c")
    base = wid * _BPW
    pltpu.sync_copy(idx_hbm.at[pl.ds(base, _BPW)], idx_v)

    lanes = jnp.arange(16, dtype=jnp.int32)

    def _scan_block(b, _):
        pltpu.sync_copy(idx_hbm.at[pl.ds(b * _SCAN, _SCAN)], scan_v)

        def _chunk(q, _):
            v = scan_v[pl.ds(q * 16, 16)]
            j = b * _SCAN + q * 16 + lanes
            plsc.store_scatter(tab, [v], j)

            def _wbody(c):
                got = plsc.load_gather(tab, [v])
                msk = got < j
                plsc.store_scatter(tab, [v], j, mask=msk)
                return jnp.max(msk.astype(jnp.int32))

            lax.while_loop(lambda c: c > 0, _wbody, jnp.int32(1))
            return 0

        return lax.fori_loop(0, _SCAN // 16, _chunk, 0)

    lax.fori_loop(0, _B // _SCAN, _scan_block, 0)

    def _wq(q, _):
        w_v[pl.ds(q * 16, 16)] = plsc.load_gather(
            tab, [idx_v[pl.ds(q * 16, 16)]])
        return 0

    lax.fori_loop(0, _BPW // 16, _wq, 0)

    for c in range(_NCH):
        ids = idx_v.at[pl.ds(c * _CH, _CH)]
        cps = [pltpu.async_copy(
                   mem_hbm.at[ids, pl.ds(src, 128)], hbuf.at[k], sem)
               for k, (src, _) in enumerate(_PIECES[:3])]
        cps.append(pltpu.async_copy(tail_hbm.at[ids], hbuf.at[3], sem))
        cp_m = pltpu.async_copy(
            rm_hbm.at[w_v.at[pl.ds(c * _CH, _CH)]], rmbuf, sem2)
        rows = pl.ds(base + c * _CH, _CH)
        for k, (_, dst) in enumerate(_PIECES):
            cps[k].wait()
            pltpu.sync_copy(hbuf.at[k], h_out.at[rows, pl.ds(dst, 128)])
        cp_m.wait()
        pltpu.sync_copy(rmbuf, rm2_out.at[rows])


@functools.cache
def _sc_gather():
    return pl.kernel(
        _sc_gather_body,
        out_type=[jax.ShapeDtypeStruct((_B, 512), jnp.float32),
                  jax.ShapeDtypeStruct((_B, 128), jnp.float32)],
        mesh=plsc.VectorSubcoreMesh(core_axis_name="c", subcore_axis_name="s",
                                    num_cores=_NC, num_subcores=_NS),
        compiler_params=pltpu.CompilerParams(needs_layout_passes=False),
        scratch_types=[
            pltpu.VMEM((_BPW,), jnp.int32),
            pltpu.VMEM((_BPW,), jnp.int32),
            pltpu.VMEM((_N,), jnp.int32),
            pltpu.VMEM((_SCAN,), jnp.int32),
            pltpu.VMEM((4, _CH, 128), jnp.float32),
            pltpu.VMEM((_CH, 128), jnp.float32),
            pltpu.SemaphoreType.DMA,
            pltpu.SemaphoreType.DMA,
        ],
    )


_BM = 256


def _tc_body(h_ref, rm_ref, w1_ref, b1_ref, w2_ref, b2_ref,
             wir_ref, wiz_ref, win_ref, bi_ref,
             whr_ref, whz_ref, whn_ref, bh_ref, out_ref):
    f32 = jnp.float32
    bf16 = jnp.bfloat16
    cdims = (((1,), (1,)), ((), ()))
    h = h_ref[:, :_D]
    hb = h.astype(bf16)
    m = jax.nn.relu(
        lax.dot_general(rm_ref[:, :_MD], w1_ref[...], cdims,
                        preferred_element_type=f32) + b1_ref[...])
    m = lax.dot_general(m, w2_ref[...], cdims,
                        preferred_element_type=f32) + b2_ref[...]
    mb = m.astype(bf16)
    gir = lax.dot_general(mb, wir_ref[...], cdims,
                          preferred_element_type=f32) + bi_ref[0:1, :]
    giz = lax.dot_general(mb, wiz_ref[...], cdims,
                          preferred_element_type=f32) + bi_ref[1:2, :]
    gin = lax.dot_general(mb, win_ref[...], cdims,
                          preferred_element_type=f32) + bi_ref[2:3, :]
    ghr = lax.dot_general(hb, whr_ref[...], cdims,
                          preferred_element_type=f32) + bh_ref[0:1, :]
    ghz = lax.dot_general(hb, whz_ref[...], cdims,
                          preferred_element_type=f32) + bh_ref[1:2, :]
    ghn = lax.dot_general(hb, whn_ref[...], cdims,
                          preferred_element_type=f32) + bh_ref[2:3, :]
    r = jax.nn.sigmoid(gir + ghr)
    z = jax.nn.sigmoid(giz + ghz)
    n = jnp.tanh(gin + r * ghn)
    out_ref[...] = (1.0 - z) * n + z * h


def _tc_call(h, rm2, W1, b1, W2, b2, W_ih, b_ih, W_hh, b_hh):
    bf16 = jnp.bfloat16
    W_ih = W_ih.astype(bf16)
    W_hh = W_hh.astype(bf16)
    wir, wiz, win = W_ih[:_D], W_ih[_D:2 * _D], W_ih[2 * _D:]
    whr, whz, whn = W_hh[:_D], W_hh[_D:2 * _D], W_hh[2 * _D:]
    bi = b_ih.reshape(3, _D)
    bh = b_hh.reshape(3, _D)
    full = lambda s: pl.BlockSpec(s, lambda i: (0, 0))
    return pl.pallas_call(
        _tc_body,
        grid=(_B // _BM,),
        in_specs=[
            pl.BlockSpec((_BM, 512), lambda i: (i, 0)),
            pl.BlockSpec((_BM, 128), lambda i: (i, 0)),
            full((_MD // 2, _MD)), full((1, _MD // 2)),
            full((_MD, _MD // 2)), full((1, _MD)),
            full((_D, _MD)), full((_D, _MD)), full((_D, _MD)), full((3, _D)),
            full((_D, _D)), full((_D, _D)), full((_D, _D)), full((3, _D)),
        ],
        out_specs=pl.BlockSpec((_BM, _D), lambda i: (i, 0)),
        out_shape=jax.ShapeDtypeStruct((_B, _D), jnp.float32),
    )(h, rm2, W1, b1.reshape(1, -1), W2, b2.reshape(1, -1),
      wir, wiz, win, bi, whr, whz, whn, bh)


def kernel(memory, node_idx, raw_messages, W1, b1, W2, b2,
           W_ih, b_ih, W_hh, b_hh):
    idx = node_idx.astype(jnp.int32)
    rm_p = jnp.pad(raw_messages, ((0, 0), (0, 128 - _MD)))
    tail = jnp.pad(lax.slice(memory, (0, 384), (_N, _D)),
                   ((0, 0), (0, 12)))
    h, rm2 = _sc_gather()(memory, tail, rm_p, idx)
    return _tc_call(h, rm2, W1, b1, W2, b2, W_ih, b_ih, W_hh, b_hh)


# speedup vs baseline: 1.1021x; 1.0551x over previous
import functools

import jax
import jax.numpy as jnp
from jax import lax
from jax.experimental import pallas as pl
from jax.experimental.pallas import tpu as pltpu
from jax.experimental.pallas import tpu_sc as plsc

_N = 100000
_D = 500
_MD = 100
_B = 16384

_NC = 2
_NS = 16
_NW = _NC * _NS
_BPW = _B // _NW
_CH = 16
_NCH = _BPW // _CH
_CRM = 32
_NRM = _BPW // _CRM
_SCAN = 1024
_NSB = _B // _SCAN


_PIECES = ((0, 0), (128, 128), (256, 256), (384, 384))


def _sc_gather_body(mem_hbm, tail_hbm, rm_hbm, idx_hbm, h_out, rm2_out,
                    idx_v, w_v, tab, scan_v, hbuf, rmbuf,
                    sem_g, sem_w, sem_s, sem_r, sem_rw):
    wid = lax.axis_index("s") * _NC + lax.axis_index("c")
    base = wid * _BPW
    pltpu.sync_copy(idx_hbm.at[pl.ds(base, _BPW)], idx_v)
    lanes = jnp.arange(16, dtype=jnp.int32)

    def _issue_h(c):
        p = c % 2
        ids = idx_v.at[pl.ds(c * _CH, _CH)]
        cps = [pltpu.async_copy(
                   mem_hbm.at[ids, pl.ds(src, 128)],
                   hbuf.at[p, :, pl.ds(dst, 128)], sem_g)
               for src, dst in _PIECES[:3]]
        cps.append(pltpu.async_copy(
            tail_hbm.at[ids], hbuf.at[p, :, pl.ds(384, 128)], sem_g))
        return cps

    def _scan_compute(b, p):
        def _chunk(q, _):
            v = scan_v[p, pl.ds(q * 16, 16)]
            j = b * _SCAN + q * 16 + lanes
            plsc.store_scatter(tab, [v], j)

            def _wbody(c):
                got = plsc.load_gather(tab, [v])
                msk = got < j
                plsc.store_scatter(tab, [v], j, mask=msk)
                return plsc.all_reduce_population_count(msk)[0]

            lax.while_loop(lambda c: c > 0, _wbody, jnp.int32(1))
            return 0

        lax.fori_loop(0, _SCAN // 16, _chunk, 0)

    g_cps = {0: _issue_h(0)}
    s_cps = {0: pltpu.async_copy(idx_hbm.at[pl.ds(0, _SCAN)],
                                 scan_v.at[0], sem_s)}
    w_cps = {}
    for c in range(_NCH):
        if c + 1 < _NCH:
            g_cps[c + 1] = _issue_h(c + 1)
        if c < _NSB:
            s_cps[c].wait()
            if c + 1 < _NSB:
                s_cps[c + 1] = pltpu.async_copy(
                    idx_hbm.at[pl.ds((c + 1) * _SCAN, _SCAN)],
                    scan_v.at[(c + 1) % 2], sem_s)
            _scan_compute(c, c % 2)
        for cp in g_cps.pop(c):
            cp.wait()
        w_cps[c] = pltpu.async_copy(
            hbuf.at[c % 2], h_out.at[pl.ds(base + c * _CH, _CH)], sem_w)
        if c - 1 in w_cps:
            w_cps.pop(c - 1).wait()
    w_cps.pop(_NCH - 1).wait()

    def _wq(q, _):
        w_v[pl.ds(q * 16, 16)] = plsc.load_gather(
            tab, [idx_v[pl.ds(q * 16, 16)]])
        return 0

    lax.fori_loop(0, _BPW // 16, _wq, 0)

    def _issue_rm(c):
        return pltpu.async_copy(
            rm_hbm.at[w_v.at[pl.ds(c * _CRM, _CRM)]], rmbuf.at[c % 2], sem_r)

    r_cps = {0: _issue_rm(0)}
    rw_cps = {}
    for c in range(_NRM):
        if c + 1 < _NRM:
            r_cps[c + 1] = _issue_rm(c + 1)
        r_cps.pop(c).wait()
        rw_cps[c] = pltpu.async_copy(
            rmbuf.at[c % 2], rm2_out.at[pl.ds(base + c * _CRM, _CRM)], sem_rw)
        if c - 1 in rw_cps:
            rw_cps.pop(c - 1).wait()
    rw_cps.pop(_NRM - 1).wait()


@functools.cache
def _sc_gather():
    return pl.kernel(
        _sc_gather_body,
        out_type=[jax.ShapeDtypeStruct((_B, 512), jnp.float32),
                  jax.ShapeDtypeStruct((_B, 128), jnp.float32)],
        mesh=plsc.VectorSubcoreMesh(core_axis_name="c", subcore_axis_name="s",
                                    num_cores=_NC, num_subcores=_NS),
        compiler_params=pltpu.CompilerParams(needs_layout_passes=False),
        scratch_types=[
            pltpu.VMEM((_BPW,), jnp.int32),
            pltpu.VMEM((_BPW,), jnp.int32),
            pltpu.VMEM((_N,), jnp.int32),
            pltpu.VMEM((2, _SCAN), jnp.int32),
            pltpu.VMEM((2, _CH, 512), jnp.float32),
            pltpu.VMEM((2, _CRM, 128), jnp.float32),
            pltpu.SemaphoreType.DMA,
            pltpu.SemaphoreType.DMA,
            pltpu.SemaphoreType.DMA,
            pltpu.SemaphoreType.DMA,
            pltpu.SemaphoreType.DMA,
        ],
    )


_BM = 256


def _tc_body(h_ref, rm_ref, w1_ref, b1_ref, w2_ref, b2_ref,
             wir_ref, wiz_ref, win_ref, bi_ref,
             whr_ref, whz_ref, whn_ref, bh_ref, out_ref):
    f32 = jnp.float32
    bf16 = jnp.bfloat16
    cdims = (((1,), (1,)), ((), ()))
    h = h_ref[:, :_D]
    hb = h.astype(bf16)
    m = jax.nn.relu(
        lax.dot_general(rm_ref[:, :_MD], w1_ref[...], cdims,
                        preferred_element_type=f32) + b1_ref[...])
    m = lax.dot_general(m, w2_ref[...], cdims,
                        preferred_element_type=f32) + b2_ref[...]
    mb = m.astype(bf16)
    gir = lax.dot_general(mb, wir_ref[...], cdims,
                          preferred_element_type=f32) + bi_ref[0:1, :]
    giz = lax.dot_general(mb, wiz_ref[...], cdims,
                          preferred_element_type=f32) + bi_ref[1:2, :]
    gin = lax.dot_general(mb, win_ref[...], cdims,
                          preferred_element_type=f32) + bi_ref[2:3, :]
    ghr = lax.dot_general(hb, whr_ref[...], cdims,
                          preferred_element_type=f32) + bh_ref[0:1, :]
    ghz = lax.dot_general(hb, whz_ref[...], cdims,
                          preferred_element_type=f32) + bh_ref[1:2, :]
    ghn = lax.dot_general(hb, whn_ref[...], cdims,
                          preferred_element_type=f32) + bh_ref[2:3, :]
    r = jax.nn.sigmoid(gir + ghr)
    z = jax.nn.sigmoid(giz + ghz)
    n = jnp.tanh(gin + r * ghn)
    out_ref[...] = (1.0 - z) * n + z * h


def _tc_call(h, rm2, W1, b1, W2, b2, W_ih, b_ih, W_hh, b_hh):
    bf16 = jnp.bfloat16
    W_ih = W_ih.astype(bf16)
    W_hh = W_hh.astype(bf16)
    wir, wiz, win = W_ih[:_D], W_ih[_D:2 * _D], W_ih[2 * _D:]
    whr, whz, whn = W_hh[:_D], W_hh[_D:2 * _D], W_hh[2 * _D:]
    bi = b_ih.reshape(3, _D)
    bh = b_hh.reshape(3, _D)
    full = lambda s: pl.BlockSpec(s, lambda i: (0, 0))
    return pl.pallas_call(
        _tc_body,
        grid=(_B // _BM,),
        in_specs=[
            pl.BlockSpec((_BM, 512), lambda i: (i, 0)),
            pl.BlockSpec((_BM, 128), lambda i: (i, 0)),
            full((_MD // 2, _MD)), full((1, _MD // 2)),
            full((_MD, _MD // 2)), full((1, _MD)),
            full((_D, _MD)), full((_D, _MD)), full((_D, _MD)), full((3, _D)),
            full((_D, _D)), full((_D, _D)), full((_D, _D)), full((3, _D)),
        ],
        out_specs=pl.BlockSpec((_BM, _D), lambda i: (i, 0)),
        out_shape=jax.ShapeDtypeStruct((_B, _D), jnp.float32),
    )(h, rm2, W1, b1.reshape(1, -1), W2, b2.reshape(1, -1),
      wir, wiz, win, bi, whr, whz, whn, bh)


def _sc_trivial_body(idx_hbm, out, buf, sem):
    wid = lax.axis_index("s") * _NC + lax.axis_index("c")
    base = wid * _BPW
    pltpu.sync_copy(idx_hbm.at[pl.ds(base, _BPW)], buf)
    pltpu.sync_copy(buf, out.at[pl.ds(base, _BPW)])


@functools.cache
def _sc_trivial():
    return pl.kernel(
        _sc_trivial_body,
        out_type=jax.ShapeDtypeStruct((_B,), jnp.int32),
        mesh=plsc.VectorSubcoreMesh(core_axis_name="c", subcore_axis_name="s",
                                    num_cores=_NC, num_subcores=_NS),
        compiler_params=pltpu.CompilerParams(needs_layout_passes=False),
        scratch_types=[
            pltpu.VMEM((_BPW,), jnp.int32),
            pltpu.SemaphoreType.DMA,
        ],
    )


def kernel(memory, node_idx, raw_messages, W1, b1, W2, b2,
           W_ih, b_ih, W_hh, b_hh):
    idx = node_idx.astype(jnp.int32)
    rm_p = jnp.pad(raw_messages, ((0, 0), (0, 128 - _MD)))
    tail = jnp.pad(lax.slice(memory, (0, 384), (_N, _D)),
                   ((0, 0), (0, 12)))
    h, rm2 = _sc_gather()(memory, tail, rm_p, idx)
    return _tc_call(h, rm2, W1, b1, W2, b2, W_ih, b_ih, W_hh, b_hh)
